# baseline (speedup 1.0000x reference)
import jax
import jax.numpy as jnp
from jax import lax
from jax.experimental import pallas as pl
from jax.experimental.pallas import tpu as pltpu
from jax.experimental.pallas import tpu_sc as plsc

N_NODES = 10000
D_FEAT = 256
HALF = D_FEAT // 2
NUM_SC = 2
NUM_TEC = 16
CHUNK = 128
NBUF = 3

ACC_ROWS = 10016
ZERO_ROWS = ACC_ROWS // NUM_TEC
OUT_ROWS = 624
TAIL_ROWS = N_NODES - NUM_TEC * OUT_ROWS


def _sc_kernel(n_chunks):
    assert n_chunks % NBUF == 0 and n_chunks > NBUF

    def body(xs_hbm, idx_hbm, zer_hbm, out_hbm,
             idx0, idx1, idx2, rows0, rows1, rows2, acc,
             zsem, isem0, isem1, isem2, gsem0, gsem1, gsem2,
             ssem0, ssem1, ssem2):
        idx_v = (idx0, idx1, idx2)
        rows = (rows0, rows1, rows2)
        isem = (isem0, isem1, isem2)
        gsem = (gsem0, gsem1, gsem2)
        ssem = (ssem0, ssem1, ssem2)
        c = lax.axis_index("c")
        s = lax.axis_index("s")
        ci0 = (c * NUM_TEC + s) * n_chunks

        cp_z = pltpu.async_copy(
            zer_hbm, acc.at[pl.ds(s * ZERO_ROWS, ZERO_ROWS)], zsem)

        def idx_start(j, b):
            pltpu.async_copy(idx_hbm.at[ci0 + j], idx_v[b], isem[b])

        def idx_wait(b):
            pltpu.make_async_copy(idx_hbm.at[0], idx_v[b], isem[b]).wait()

        def gather_start(b):
            pltpu.async_copy(xs_hbm.at[idx_v[b].at[0]], rows[b], gsem[b])

        def gather_wait(b):
            pltpu.make_async_copy(xs_hbm.at[idx_v[b].at[0]], rows[b],
                                  gsem[b]).wait()

        def scatter_start(b):
            pltpu.async_copy(rows[b], acc.at[idx_v[b].at[1]], ssem[b],
                             add=True)

        def scatter_wait(b):
            pltpu.make_async_copy(rows[b], acc.at[idx_v[b].at[1]],
                                  ssem[b]).wait()

        idx_start(0, 0)
        idx_wait(0)
        gather_start(0)
        idx_start(1, 1)
        cp_z.wait()
        plsc.subcore_barrier()
        idx_wait(1)
        gather_start(1)
        gather_wait(0)
        scatter_start(0)
        idx_start(2, 2)

        def outer(o, carry):
            j0 = NBUF + o * NBUF
            for b in range(NBUF):
                j = j0 + b
                b1 = (b + NBUF - 1) % NBUF
                idx_wait(b1)
                gather_start(b1)
                b2 = (b + NBUF - 2) % NBUF
                gather_wait(b2)
                scatter_start(b2)
                scatter_wait(b)
                idx_start(j, b)
            return carry

        lax.fori_loop(0, (n_chunks - NBUF) // NBUF, outer, 0)

        bl = (n_chunks - 1) % NBUF
        idx_wait(bl)
        gather_start(bl)
        gather_wait((bl + NBUF - 1) % NBUF)
        scatter_start((bl + NBUF - 1) % NBUF)
        gather_wait(bl)
        scatter_start(bl)
        for b in range(NBUF):
            scatter_wait(b)
        plsc.subcore_barrier()

        pltpu.sync_copy(
            acc.at[pl.ds(s * OUT_ROWS, OUT_ROWS)],
            out_hbm.at[pl.ds(s * OUT_ROWS, OUT_ROWS), pl.ds(c * HALF, HALF)])

        @pl.when(s == NUM_TEC - 1)
        def _tail():
            r0 = NUM_TEC * OUT_ROWS
            pltpu.sync_copy(
                acc.at[pl.ds(r0, TAIL_ROWS)],
                out_hbm.at[pl.ds(r0, TAIL_ROWS), pl.ds(c * HALF, HALF)])

    mesh = plsc.VectorSubcoreMesh(core_axis_name="c", subcore_axis_name="s")
    return pl.kernel(
        body,
        out_type=jax.ShapeDtypeStruct((N_NODES, D_FEAT), jnp.float32),
        mesh=mesh,
        scratch_types=(
            [pltpu.VMEM((2, CHUNK), jnp.int32)] * NBUF
            + [pltpu.VMEM((CHUNK, HALF), jnp.float32)] * NBUF
            + [pltpu.VMEM_SHARED((ACC_ROWS, HALF), jnp.float32)]
            + [pltpu.SemaphoreType.DMA] * (1 + 3 * NBUF)
        ),
    )


@jax.jit
def kernel(x, up_index, down_index):
    n_edges = up_index.shape[1] + down_index.shape[1]
    align = NUM_TEC * CHUNK * NBUF
    e_pad = ((n_edges + align - 1) // align) * align
    n_chunks = e_pad // (NUM_TEC * CHUNK)
    pad = e_pad - n_edges

    src = jnp.concatenate(
        [up_index[0], down_index[0], jnp.zeros((pad,), up_index.dtype)]
    ).astype(jnp.int32)
    dst = jnp.concatenate(
        [up_index[1], down_index[1],
         jnp.full((pad,), N_NODES, up_index.dtype)]
    ).astype(jnp.int32)
    src_all = jnp.concatenate([src, src + N_NODES]).reshape(-1, 1, CHUNK)
    dst_all = jnp.concatenate([dst, dst]).reshape(-1, 1, CHUNK)
    idx_all = jnp.concatenate([src_all, dst_all], axis=1)
    xs = jnp.concatenate([x[:, :HALF], x[:, HALF:]], axis=0)
    zer = jnp.zeros((ZERO_ROWS, HALF), jnp.float32)

    return _sc_kernel(n_chunks)(xs, idx_all, zer)
